# baseline (device time: 12108 ns/iter reference)
import jax
import jax.numpy as jnp
from jax import lax
from jax.experimental import pallas as pl
from jax.experimental.pallas import tpu as pltpu

N_DEV = 4
N_CHUNKS = 8
N_BUF = 4
N_HALF = 2


def kernel(x):
    m, n = x.shape
    block_m = m // N_CHUNKS
    nc = n // N_HALF

    def body(
        x_hbm,
        out_hbm,
        chunk_buf,
        comm_ref,
        out_vmem,
        copy_sems,
        send_sems,
        recv_sems,
        out_sem,
    ):
        my = lax.axis_index("i")
        barrier_sem = pltpu.get_barrier_semaphore()

        for off in range(1, N_DEV):
            pl.semaphore_signal(
                barrier_sem,
                inc=1,
                device_id=((my + off) % N_DEV,),
                device_id_type=pl.DeviceIdType.MESH,
            )

        def start_comm(h, partial):
            comm_ref[h, N_DEV - 1] = partial
            rdmas = []
            for off in range(1, N_DEV):
                rdma = pltpu.make_async_remote_copy(
                    src_ref=comm_ref.at[h, N_DEV - 1],
                    dst_ref=comm_ref.at[h, off - 1],
                    send_sem=send_sems.at[h, off - 1],
                    recv_sem=recv_sems.at[h, off - 1],
                    device_id=((my + off) % N_DEV,),
                    device_id_type=pl.DeviceIdType.MESH,
                )
                rdma.start()
                rdmas.append(rdma)
            return rdmas

        total = N_HALF * N_CHUNKS

        def chunk_copy(c):
            h, r = divmod(c, N_CHUNKS)
            return pltpu.make_async_copy(
                x_hbm.at[pl.ds(r * block_m, block_m), pl.ds(h * nc, nc)],
                chunk_buf.at[c % N_BUF],
                copy_sems.at[c % N_BUF],
            )

        copies = [chunk_copy(c) for c in range(total)]
        for k in range(N_BUF - 1):
            copies[k].start()
        accs = [None, None]
        comm0 = None
        for c in range(total):
            if c + N_BUF - 1 < total:
                copies[c + N_BUF - 1].start()
            copies[c].wait()
            h = c // N_CHUNKS
            part = jnp.sum(chunk_buf[c % N_BUF], axis=0, keepdims=True)
            accs[h] = part if accs[h] is None else accs[h] + part
            if c == N_CHUNKS - 1:
                pl.semaphore_wait(barrier_sem, N_DEV - 1)
                comm0 = start_comm(0, accs[0])

        comm1 = start_comm(1, accs[1])
        for rdma in comm0:
            rdma.wait()
        out_vmem[:, 0:nc] = (
            comm_ref[0, 0] + comm_ref[0, 1] + comm_ref[0, 2] + comm_ref[0, 3]
        )
        for rdma in comm1:
            rdma.wait()
        out_vmem[:, nc:n] = (
            comm_ref[1, 0] + comm_ref[1, 1] + comm_ref[1, 2] + comm_ref[1, 3]
        )

        out_copy = pltpu.make_async_copy(out_vmem, out_hbm, out_sem)
        out_copy.start()
        out_copy.wait()

    x = pltpu.with_memory_space_constraint(x, pltpu.MemorySpace.HBM)

    return pl.pallas_call(
        body,
        out_shape=jax.ShapeDtypeStruct((1, n), x.dtype),
        in_specs=[pl.BlockSpec(memory_space=pl.ANY)],
        out_specs=pl.BlockSpec(memory_space=pl.ANY),
        scratch_shapes=[
            pltpu.VMEM((N_BUF, block_m, nc), x.dtype),
            pltpu.VMEM((N_HALF, N_DEV, 1, nc), x.dtype),
            pltpu.VMEM((1, n), x.dtype),
            pltpu.SemaphoreType.DMA((N_BUF,)),
            pltpu.SemaphoreType.DMA((N_HALF, N_DEV - 1)),
            pltpu.SemaphoreType.DMA((N_HALF, N_DEV - 1)),
            pltpu.SemaphoreType.DMA,
        ],
        compiler_params=pltpu.CompilerParams(collective_id=0),
    )(x)


# device time: 11104 ns/iter; 1.0904x vs baseline; 1.0904x over previous
import jax
import jax.numpy as jnp
from jax import lax
from jax.experimental import pallas as pl
from jax.experimental.pallas import tpu as pltpu

N_DEV = 4
N_CHUNKS = 8
N_BUF = 4


def kernel(x):
    m, n = x.shape
    block_m = m // N_CHUNKS

    def body(
        x_hbm,
        out_hbm,
        chunk_buf,
        comm_ref,
        out_vmem,
        copy_sems,
        send_sems,
        recv_sems,
        out_sem,
    ):
        my = lax.axis_index("i")
        barrier_sem = pltpu.get_barrier_semaphore()

        copies = [
            pltpu.make_async_copy(
                x_hbm.at[pl.ds(c * block_m, block_m), :],
                chunk_buf.at[c % N_BUF],
                copy_sems.at[c % N_BUF],
            )
            for c in range(N_CHUNKS)
        ]
        for k in range(N_BUF - 1):
            copies[k].start()

        for off in range(1, N_DEV):
            pl.semaphore_signal(
                barrier_sem,
                inc=1,
                device_id=((my + off) % N_DEV,),
                device_id_type=pl.DeviceIdType.MESH,
            )
        pl.semaphore_wait(barrier_sem, N_DEV - 1)

        acc = None
        for c in range(N_CHUNKS):
            if c + N_BUF - 1 < N_CHUNKS:
                copies[c + N_BUF - 1].start()
            copies[c].wait()
            part = jnp.sum(chunk_buf[c % N_BUF], axis=0, keepdims=True)
            acc = part if acc is None else acc + part

        comm_ref[N_DEV - 1] = acc

        rdmas = []
        for off in range(1, N_DEV):
            rdma = pltpu.make_async_remote_copy(
                src_ref=comm_ref.at[N_DEV - 1],
                dst_ref=comm_ref.at[off - 1],
                send_sem=send_sems.at[off - 1],
                recv_sem=recv_sems.at[off - 1],
                device_id=((my + off) % N_DEV,),
                device_id_type=pl.DeviceIdType.MESH,
            )
            rdma.start()
            rdmas.append(rdma)

        total = acc
        for off in (1, 3, 2):
            rdmas[off - 1].wait()
            total = total + comm_ref[off - 1]
        out_vmem[:, :] = total

        out_copy = pltpu.make_async_copy(out_vmem, out_hbm, out_sem)
        out_copy.start()
        out_copy.wait()

    x = pltpu.with_memory_space_constraint(x, pltpu.MemorySpace.HBM)

    return pl.pallas_call(
        body,
        out_shape=jax.ShapeDtypeStruct((1, n), x.dtype),
        in_specs=[pl.BlockSpec(memory_space=pl.ANY)],
        out_specs=pl.BlockSpec(memory_space=pl.ANY),
        scratch_shapes=[
            pltpu.VMEM((N_BUF, block_m, n), x.dtype),
            pltpu.VMEM((N_DEV, 1, n), x.dtype),
            pltpu.VMEM((1, n), x.dtype),
            pltpu.SemaphoreType.DMA((N_BUF,)),
            pltpu.SemaphoreType.DMA((N_DEV - 1,)),
            pltpu.SemaphoreType.DMA((N_DEV - 1,)),
            pltpu.SemaphoreType.DMA,
        ],
        compiler_params=pltpu.CompilerParams(collective_id=0),
    )(x)
